# baseline (speedup 1.0000x reference)
import jax
import jax.numpy as jnp
from jax import lax
from jax.experimental import pallas as pl
from jax.experimental.pallas import tpu as pltpu
from jax.experimental.pallas import tpu_sc as plsc
import functools

B = 128
VOCAB = 100000
L = 16
NCHUNK = B // L


@functools.partial(
    pl.kernel,
    out_type=jax.ShapeDtypeStruct((L,), jnp.float32),
    mesh=plsc.VectorSubcoreMesh(core_axis_name="c", subcore_axis_name="s"),
    compiler_params=pltpu.CompilerParams(needs_layout_passes=False),
    scratch_types=[
        pltpu.VMEM((B,), jnp.int32),
        pltpu.VMEM((B,), jnp.int32),
        pltpu.VMEM((B,), jnp.float32),
        pltpu.VMEM((L,), jnp.float32),
        pltpu.SemaphoreType.DMA,
    ],
)
def _logit_loss_sc(flat_hbm, labels_hbm, out_hbm, lab_v, idx_v, val_v, res_v, sem):
    tile0 = jnp.logical_and(lax.axis_index("c") == 0, lax.axis_index("s") == 0)

    @pl.when(tile0)
    def _():
        pltpu.sync_copy(labels_hbm, lab_v)
        lane = lax.iota(jnp.int32, L)
        for c in range(NCHUNK):
            rows = c * L + lane
            idx_v[pl.ds(c * L, L)] = lab_v[pl.ds(c * L, L)] + rows * VOCAB
        pltpu.async_copy(flat_hbm.at[idx_v], val_v, sem).wait()
        acc = val_v[pl.ds(0, L)]
        for c in range(1, NCHUNK):
            acc = acc + val_v[pl.ds(c * L, L)]
        for shift in (8, 4, 2, 1):
            res_v[...] = acc
            acc = acc + plsc.load_gather(res_v, [lane ^ shift])
        res_v[...] = acc
        pltpu.sync_copy(res_v, out_hbm)


def kernel(logits, labels):
    flat = logits.reshape(-1)
    out = _logit_loss_sc(flat, labels.astype(jnp.int32))
    return out[0]
